# baseline (speedup 1.0000x reference)
import functools

import jax
import jax.numpy as jnp
from jax import lax
from jax.experimental import pallas as pl
from jax.experimental.pallas import tpu as pltpu
from jax.experimental.pallas import tpu_sc as plsc

NUM_EMB = 1000000
DIM = 64
TOKENS = 16384 * 20
NUM_CORES = 2
NUM_SUBCORES = 16
NW = NUM_CORES * NUM_SUBCORES
ROWS_PER_W = TOKENS // NW
CHUNK = 128
NCHUNK = ROWS_PER_W // CHUNK


def _make_embedding_kernel():
    mesh = plsc.VectorSubcoreMesh(core_axis_name="c", subcore_axis_name="s")

    @functools.partial(
        pl.kernel,
        mesh=mesh,
        compiler_params=pltpu.CompilerParams(use_tc_tiling_on_sc=False),
        out_type=jax.ShapeDtypeStruct((TOKENS, DIM), jnp.float32),
        scratch_types=[
            pltpu.VMEM((NCHUNK, CHUNK), jnp.int32),
            pltpu.VMEM((CHUNK, DIM), jnp.float32),
            pltpu.SemaphoreType.DMA,
        ],
    )
    def emb(tok_hbm, w_hbm, out_hbm, idx_v, rows_v, sem):
        wid = lax.axis_index("s") * NUM_CORES + lax.axis_index("c")
        base = wid * ROWS_PER_W
        pltpu.sync_copy(tok_hbm.at[wid], idx_v)

        def step(j, carry):
            pltpu.async_copy(w_hbm.at[idx_v.at[j]], rows_v, sem).wait()
            pltpu.sync_copy(rows_v, out_hbm.at[pl.ds(base + j * CHUNK, CHUNK)])
            return carry

        lax.fori_loop(0, NCHUNK, step, 0)

    return emb


_emb = _make_embedding_kernel()


def kernel(token_ids, weight):
    tok = token_ids.reshape(NW, NCHUNK, CHUNK)
    out = _emb(tok, weight)
    return out.reshape(16384, 20, DIM)


# speedup vs baseline: 1.0216x; 1.0216x over previous
import functools

import jax
import jax.numpy as jnp
from jax import lax
from jax.experimental import pallas as pl
from jax.experimental.pallas import tpu as pltpu
from jax.experimental.pallas import tpu_sc as plsc

NUM_EMB = 1000000
DIM = 64
TOKENS = 16384 * 20
NUM_CORES = 2
NUM_SUBCORES = 16
NW = NUM_CORES * NUM_SUBCORES
ROWS_PER_W = TOKENS // NW
CHUNK = 128
NCHUNK = ROWS_PER_W // CHUNK


def _make_embedding_kernel():
    mesh = plsc.VectorSubcoreMesh(core_axis_name="c", subcore_axis_name="s")

    @functools.partial(
        pl.kernel,
        mesh=mesh,
        compiler_params=pltpu.CompilerParams(use_tc_tiling_on_sc=False),
        out_type=jax.ShapeDtypeStruct((TOKENS, DIM), jnp.float32),
        scratch_types=[
            pltpu.VMEM((ROWS_PER_W,), jnp.int32),
            pltpu.VMEM((2, CHUNK, DIM), jnp.float32),
            pltpu.SemaphoreType.DMA,
            pltpu.SemaphoreType.DMA,
        ],
    )
    def emb(tok_hbm, w_hbm, out_hbm, idx_v, rows_v, sem_g, sem_s):
        wid = lax.axis_index("s") * NUM_CORES + lax.axis_index("c")
        base = wid * ROWS_PER_W
        pltpu.sync_copy(tok_hbm.at[pl.ds(base, ROWS_PER_W)], idx_v)

        def gather_start(j, b):
            pltpu.make_async_copy(
                w_hbm.at[idx_v.at[pl.ds(j * CHUNK, CHUNK)]], rows_v.at[b], sem_g
            ).start()

        def gather_wait(b):
            pltpu.make_async_copy(
                w_hbm.at[idx_v.at[pl.ds(0, CHUNK)]], rows_v.at[b], sem_g
            ).wait()

        def store_start(j, b):
            pltpu.make_async_copy(
                rows_v.at[b], out_hbm.at[pl.ds(base + j * CHUNK, CHUNK)], sem_s
            ).start()

        def store_wait(b):
            pltpu.make_async_copy(
                rows_v.at[b], out_hbm.at[pl.ds(base, CHUNK)], sem_s
            ).wait()

        gather_start(0, 0)

        def step(j, carry):
            b = lax.rem(j, 2)
            gather_wait(b)

            @pl.when(j > 0)
            def _():
                store_wait(1 - b)

            @pl.when(j + 1 < NCHUNK)
            def _():
                gather_start(j + 1, 1 - b)

            store_start(j, b)
            return carry

        lax.fori_loop(0, NCHUNK, step, 0)
        store_wait(lax.rem(NCHUNK - 1, 2))

    return emb


_emb = _make_embedding_kernel()


def kernel(token_ids, weight):
    tok = token_ids.reshape(TOKENS)
    out = _emb(tok, weight)
    return out.reshape(16384, 20, DIM)
